# baseline (speedup 1.0000x reference)
import functools

import jax
import jax.numpy as jnp
from jax import lax
from jax.experimental import pallas as pl
from jax.experimental.pallas import tpu as pltpu
from jax.experimental.pallas import tpu_sc as plsc

_NF = 1_000_000
_B = 16384
_D = 32
_NC = 2
_NS = 16
_NW = _NC * _NS
_BPW = _B // _NW
_W = _D * _BPW

_mesh = plsc.VectorSubcoreMesh(core_axis_name="c", subcore_axis_name="s")


@functools.partial(
    pl.kernel,
    mesh=_mesh,
    out_type=(
        jax.ShapeDtypeStruct((_B,), jnp.float32),
        jax.ShapeDtypeStruct((_D * _B,), jnp.float32),
    ),
    scratch_types=[
        pltpu.VMEM((_BPW,), jnp.int32),
        pltpu.VMEM((_W,), jnp.float32),
        pltpu.VMEM((_BPW,), jnp.float32),
        pltpu.SemaphoreType.DMA,
        pltpu.SemaphoreType.DMA,
    ],
    compiler_params=pltpu.CompilerParams(use_tc_tiling_on_sc=False),
)
def _emb(idx_hbm, vt_hbm, bias_hbm, out_b, out_v,
         idx_v, vbuf, bb, sem_v, sem_b):
    wid = lax.axis_index("s") * _NC + lax.axis_index("c")
    base = wid * _BPW
    pltpu.sync_copy(idx_hbm.at[pl.ds(base, _BPW)], idx_v)

    bias_cp = pltpu.async_copy(bias_hbm.at[idx_v], bb, sem_b)
    copies = [
        pltpu.async_copy(
            vt_hbm.at[d].at[idx_v], vbuf.at[pl.ds(d * _BPW, _BPW)], sem_v)
        for d in range(_D)
    ]

    bias_cp.wait()
    pltpu.sync_copy(bb, out_b.at[pl.ds(base, _BPW)])

    for d in range(_D):
        copies[d].wait()
        pltpu.sync_copy(
            vbuf.at[pl.ds(d * _BPW, _BPW)],
            out_v.at[pl.ds(d * _B + base, _BPW)],
        )


def kernel(index, vect, bias):
    idx = index.astype(jnp.int32)
    vt = vect.T
    bflat = bias.reshape(-1)
    out_b, out_v = _emb(idx, vt, bflat)
    return (out_b, out_v.reshape(_D, _B).T)
